# baseline (speedup 1.0000x reference)
import jax
import jax.numpy as jnp
from jax.experimental import pallas as pl
from jax.experimental.pallas import tpu as pltpu

STEP_SIZE = 0.1
MAX_RESAMPLINGS = 50
N_ROWS = 16384
N_COLS = 128
BLOCK_ROWS = 1024
T1 = 12


def _threefry2x32(k0, k1, x0, x1):
    def rotl(v, r):
        return (v << jnp.uint32(r)) | (v >> jnp.uint32(32 - r))

    ks2 = k0 ^ k1 ^ jnp.uint32(0x1BD11BDA)
    rots = ((13, 15, 26, 6), (17, 29, 16, 24))
    inject = ((k1, ks2, 1), (ks2, k0, 2), (k0, k1, 3), (k1, ks2, 4), (ks2, k0, 5))
    x0 = x0 + k0
    x1 = x1 + k1
    for g in range(5):
        for r in rots[g % 2]:
            x0 = x0 + x1
            x1 = rotl(x1, r)
            x1 = x0 ^ x1
        a, b, c = inject[g]
        x0 = x0 + a
        x1 = x1 + b + jnp.uint32(c)
    return x0, x1


def _uniform(k0, k1, idx):
    o0, o1 = _threefry2x32(k0, k1, jnp.zeros_like(idx), idx)
    bits = o0 ^ o1
    f = jax.lax.bitcast_convert_type(
        (bits >> jnp.uint32(9)) | jnp.uint32(0x3F800000), jnp.float32
    )
    return f - 1.0


def _one_iter(kd_ref, i, x, y, flat_ids, a, b, x_new, rej_f):
    reject = rej_f > 0.0
    smaller = x_new < x
    a = jnp.where(reject & smaller, x_new, a)
    b = jnp.where(reject & (~smaller), x_new, b)
    u = _uniform(kd_ref[4 + i, 0], kd_ref[4 + i, 1], flat_ids)
    x_new = jnp.where(reject, u * (b - a) + a, x_new)
    pot = -0.5 * jnp.sum(x_new * x_new, axis=-1, keepdims=True)
    rej_f = jnp.where(reject & (pot < y), 1.0, 0.0).astype(jnp.float32)
    return a, b, x_new, rej_f


def _phase_a(kd_ref, x_ref, xn_ref, a_ref, b_ref, y_ref, rej_ref):
    i_blk = pl.program_id(0)
    x = x_ref[...]
    rows, cols = x.shape

    row_in_blk = jax.lax.broadcasted_iota(jnp.uint32, (rows, 1), 0)
    row_ids = jnp.uint32(rows) * jnp.uint32(i_blk) + row_in_blk
    col_ids = jax.lax.broadcasted_iota(jnp.uint32, (rows, cols), 1)
    flat_ids = row_ids * jnp.uint32(cols) + col_ids

    pot_x = -0.5 * jnp.sum(x * x, axis=-1, keepdims=True)

    k0, k1 = kd_ref[0, 0], kd_ref[0, 1]
    y = jnp.log(1.0 - _uniform(k0, k1, row_ids)) + pot_x

    k0, k1 = kd_ref[2, 0], kd_ref[2, 1]
    sw = jnp.log(1.0 - _uniform(k0, k1, flat_ids)) / (-STEP_SIZE)
    a = x - sw / 2.0
    b = x + sw / 2.0

    k0, k1 = kd_ref[3, 0], kd_ref[3, 1]
    x_new = _uniform(k0, k1, flat_ids) * (b - a) + a
    pot = -0.5 * jnp.sum(x_new * x_new, axis=-1, keepdims=True)
    rej_f = jnp.where(pot < y, 1.0, 0.0).astype(jnp.float32)

    def cond(carry):
        i, _, _, _, rej_f = carry
        return jnp.logical_and(i < T1, jnp.sum(rej_f) > 0.0)

    def step(carry):
        i, a, b, x_new, rej_f = carry
        a, b, x_new, rej_f = _one_iter(kd_ref, i, x, y, flat_ids, a, b, x_new, rej_f)
        a, b, x_new, rej_f = _one_iter(
            kd_ref, i + 1, x, y, flat_ids, a, b, x_new, rej_f
        )
        return i + 2, a, b, x_new, rej_f

    _, a, b, x_new, rej_f = jax.lax.while_loop(
        cond, step, (jnp.int32(0), a, b, x_new, rej_f)
    )
    xn_ref[...] = x_new
    a_ref[...] = a
    b_ref[...] = b
    y_ref[...] = y
    rej_ref[...] = rej_f


def _phase_b(kd_ref, x_ref, xn_in_ref, a_ref, b_ref, y_ref, rej_ref, rid_ref,
             xn_out_ref):
    x = x_ref[...]
    rows, cols = x.shape
    row_ids = rid_ref[...].astype(jnp.uint32)
    col_ids = jax.lax.broadcasted_iota(jnp.uint32, (rows, cols), 1)
    flat_ids = row_ids * jnp.uint32(cols) + col_ids

    y = y_ref[...]
    a = a_ref[...]
    b = b_ref[...]
    x_new = xn_in_ref[...]
    rej_f = rej_ref[...]

    def cond(carry):
        i, _, _, _, rej_f = carry
        return jnp.logical_and(i < MAX_RESAMPLINGS, jnp.sum(rej_f) > 0.0)

    def step(carry):
        i, a, b, x_new, rej_f = carry
        a, b, x_new, rej_f = _one_iter(kd_ref, i, x, y, flat_ids, a, b, x_new, rej_f)
        a, b, x_new, rej_f = _one_iter(
            kd_ref, i + 1, x, y, flat_ids, a, b, x_new, rej_f
        )
        return i + 2, a, b, x_new, rej_f

    _, _, _, x_new, _ = jax.lax.while_loop(
        cond, step, (jnp.int32(T1), a, b, x_new, rej_f)
    )
    xn_out_ref[...] = x_new


def _row_spec():
    return pl.BlockSpec((BLOCK_ROWS, N_COLS), lambda i: (i, 0))


def _col_spec():
    return pl.BlockSpec((BLOCK_ROWS, 1), lambda i: (i, 0))


@jax.jit
def kernel(x):
    ks = jax.random.split(jax.random.key(1), 4 + MAX_RESAMPLINGS)
    kd = jax.random.key_data(ks).astype(jnp.uint32)

    mat = jax.ShapeDtypeStruct((N_ROWS, N_COLS), jnp.float32)
    col = jax.ShapeDtypeStruct((N_ROWS, 1), jnp.float32)

    x_new, a, b, y, rej = pl.pallas_call(
        _phase_a,
        grid=(N_ROWS // BLOCK_ROWS,),
        in_specs=[pl.BlockSpec(memory_space=pltpu.SMEM), _row_spec()],
        out_specs=[_row_spec(), _row_spec(), _row_spec(), _col_spec(), _col_spec()],
        out_shape=[mat, mat, mat, col, col],
        compiler_params=pltpu.CompilerParams(
            dimension_semantics=("arbitrary",),
        ),
    )(kd, x)

    flags = (rej[:, 0] > 0.0).astype(jnp.int32)
    csum = jnp.cumsum(flags)
    n_active = csum[-1]
    iota = jnp.arange(N_ROWS, dtype=jnp.int32)
    inv = jnp.where(flags > 0, csum - 1, n_active + iota - csum).astype(jnp.int32)
    order = jnp.zeros((N_ROWS,), jnp.int32).at[inv].set(iota)

    xs = x[order]
    xns = x_new[order]
    as_ = a[order]
    bs = b[order]
    ys = y[order]
    rejs = rej[order]
    rid = order[:, None]

    xn_final = pl.pallas_call(
        _phase_b,
        grid=(N_ROWS // BLOCK_ROWS,),
        in_specs=[
            pl.BlockSpec(memory_space=pltpu.SMEM),
            _row_spec(), _row_spec(), _row_spec(), _row_spec(),
            _col_spec(), _col_spec(),
            pl.BlockSpec((BLOCK_ROWS, 1), lambda i: (i, 0)),
        ],
        out_specs=_row_spec(),
        out_shape=mat,
        compiler_params=pltpu.CompilerParams(
            dimension_semantics=("arbitrary",),
        ),
    )(kd, xs, xns, as_, bs, ys, rejs, rid)

    return xn_final[inv]


# speedup vs baseline: 1.0574x; 1.0574x over previous
import jax
import jax.numpy as jnp
from jax.experimental import pallas as pl
from jax.experimental.pallas import tpu as pltpu

STEP_SIZE = 0.1
MAX_RESAMPLINGS = 50
N_ROWS = 16384
N_COLS = 128
BLOCK_ROWS = 1024
T1 = 12
B_BLOCK_ROWS = 512


def _threefry2x32(k0, k1, x0, x1):
    def rotl(v, r):
        return (v << jnp.uint32(r)) | (v >> jnp.uint32(32 - r))

    ks2 = k0 ^ k1 ^ jnp.uint32(0x1BD11BDA)
    rots = ((13, 15, 26, 6), (17, 29, 16, 24))
    inject = ((k1, ks2, 1), (ks2, k0, 2), (k0, k1, 3), (k1, ks2, 4), (ks2, k0, 5))
    x0 = x0 + k0
    x1 = x1 + k1
    for g in range(5):
        for r in rots[g % 2]:
            x0 = x0 + x1
            x1 = rotl(x1, r)
            x1 = x0 ^ x1
        a, b, c = inject[g]
        x0 = x0 + a
        x1 = x1 + b + jnp.uint32(c)
    return x0, x1


def _uniform(k0, k1, idx):
    o0, o1 = _threefry2x32(k0, k1, jnp.zeros_like(idx), idx)
    bits = o0 ^ o1
    f = jax.lax.bitcast_convert_type(
        (bits >> jnp.uint32(9)) | jnp.uint32(0x3F800000), jnp.float32
    )
    return f - 1.0


def _one_iter(kd_ref, i, x, y, flat_ids, a, b, x_new, rej_f):
    reject = rej_f > 0.0
    smaller = x_new < x
    a = jnp.where(reject & smaller, x_new, a)
    b = jnp.where(reject & (~smaller), x_new, b)
    u = _uniform(kd_ref[4 + i, 0], kd_ref[4 + i, 1], flat_ids)
    x_new = jnp.where(reject, u * (b - a) + a, x_new)
    pot = -0.5 * jnp.sum(x_new * x_new, axis=-1, keepdims=True)
    rej_f = jnp.where(reject & (pot < y), 1.0, 0.0).astype(jnp.float32)
    return a, b, x_new, rej_f


def _phase_a(kd_ref, x_ref, xn_ref, a_ref, b_ref, y_ref, rej_ref):
    i_blk = pl.program_id(0)
    x = x_ref[...]
    rows, cols = x.shape

    row_in_blk = jax.lax.broadcasted_iota(jnp.uint32, (rows, 1), 0)
    row_ids = jnp.uint32(rows) * jnp.uint32(i_blk) + row_in_blk
    col_ids = jax.lax.broadcasted_iota(jnp.uint32, (rows, cols), 1)
    flat_ids = row_ids * jnp.uint32(cols) + col_ids

    pot_x = -0.5 * jnp.sum(x * x, axis=-1, keepdims=True)

    k0, k1 = kd_ref[0, 0], kd_ref[0, 1]
    y = jnp.log(1.0 - _uniform(k0, k1, row_ids)) + pot_x

    k0, k1 = kd_ref[2, 0], kd_ref[2, 1]
    sw = jnp.log(1.0 - _uniform(k0, k1, flat_ids)) / (-STEP_SIZE)
    a = x - sw / 2.0
    b = x + sw / 2.0

    k0, k1 = kd_ref[3, 0], kd_ref[3, 1]
    x_new = _uniform(k0, k1, flat_ids) * (b - a) + a
    pot = -0.5 * jnp.sum(x_new * x_new, axis=-1, keepdims=True)
    rej_f = jnp.where(pot < y, 1.0, 0.0).astype(jnp.float32)

    def cond(carry):
        i, _, _, _, rej_f = carry
        return jnp.logical_and(i < T1, jnp.sum(rej_f) > 0.0)

    def step(carry):
        i, a, b, x_new, rej_f = carry
        a, b, x_new, rej_f = _one_iter(kd_ref, i, x, y, flat_ids, a, b, x_new, rej_f)
        a, b, x_new, rej_f = _one_iter(
            kd_ref, i + 1, x, y, flat_ids, a, b, x_new, rej_f
        )
        return i + 2, a, b, x_new, rej_f

    _, a, b, x_new, rej_f = jax.lax.while_loop(
        cond, step, (jnp.int32(0), a, b, x_new, rej_f)
    )
    xn_ref[...] = x_new
    a_ref[...] = a
    b_ref[...] = b
    y_ref[...] = y
    rej_ref[...] = rej_f


def _phase_b(kd_ref, x_ref, xn_in_ref, a_ref, b_ref, y_ref, rej_ref, rid_ref,
             xn_out_ref):
    x = x_ref[...]
    rows, cols = x.shape
    row_ids = rid_ref[...].astype(jnp.uint32)
    col_ids = jax.lax.broadcasted_iota(jnp.uint32, (rows, cols), 1)
    flat_ids = row_ids * jnp.uint32(cols) + col_ids

    y = y_ref[...]
    a = a_ref[...]
    b = b_ref[...]
    x_new = xn_in_ref[...]
    rej_f = rej_ref[...]

    def cond(carry):
        i, _, _, _, rej_f = carry
        return jnp.logical_and(i < MAX_RESAMPLINGS, jnp.sum(rej_f) > 0.0)

    def step(carry):
        i, a, b, x_new, rej_f = carry
        a, b, x_new, rej_f = _one_iter(kd_ref, i, x, y, flat_ids, a, b, x_new, rej_f)
        a, b, x_new, rej_f = _one_iter(
            kd_ref, i + 1, x, y, flat_ids, a, b, x_new, rej_f
        )
        return i + 2, a, b, x_new, rej_f

    _, _, _, x_new, _ = jax.lax.while_loop(
        cond, step, (jnp.int32(T1), a, b, x_new, rej_f)
    )
    xn_out_ref[...] = x_new


def _row_spec(rows=BLOCK_ROWS):
    return pl.BlockSpec((rows, N_COLS), lambda i: (i, 0))


def _col_spec(rows=BLOCK_ROWS):
    return pl.BlockSpec((rows, 1), lambda i: (i, 0))


@jax.jit
def kernel(x):
    ks = jax.random.split(jax.random.key(1), 4 + MAX_RESAMPLINGS)
    kd = jax.random.key_data(ks).astype(jnp.uint32)

    mat = jax.ShapeDtypeStruct((N_ROWS, N_COLS), jnp.float32)
    col = jax.ShapeDtypeStruct((N_ROWS, 1), jnp.float32)

    x_new, a, b, y, rej = pl.pallas_call(
        _phase_a,
        grid=(N_ROWS // BLOCK_ROWS,),
        in_specs=[pl.BlockSpec(memory_space=pltpu.SMEM), _row_spec()],
        out_specs=[_row_spec(), _row_spec(), _row_spec(), _col_spec(), _col_spec()],
        out_shape=[mat, mat, mat, col, col],
        compiler_params=pltpu.CompilerParams(
            dimension_semantics=("arbitrary",),
        ),
    )(kd, x)

    order = jnp.argsort(-rej[:, 0]).astype(jnp.int32)
    inv = jnp.argsort(order).astype(jnp.int32)

    xs = x[order]
    xns = x_new[order]
    as_ = a[order]
    bs = b[order]
    ys = y[order]
    rejs = rej[order]
    rid = order[:, None]

    xn_final = pl.pallas_call(
        _phase_b,
        grid=(N_ROWS // B_BLOCK_ROWS,),
        in_specs=[
            pl.BlockSpec(memory_space=pltpu.SMEM),
            _row_spec(B_BLOCK_ROWS), _row_spec(B_BLOCK_ROWS),
            _row_spec(B_BLOCK_ROWS), _row_spec(B_BLOCK_ROWS),
            _col_spec(B_BLOCK_ROWS), _col_spec(B_BLOCK_ROWS),
            pl.BlockSpec((B_BLOCK_ROWS, 1), lambda i: (i, 0)),
        ],
        out_specs=_row_spec(B_BLOCK_ROWS),
        out_shape=mat,
        compiler_params=pltpu.CompilerParams(
            dimension_semantics=("arbitrary",),
        ),
    )(kd, xs, xns, as_, bs, ys, rejs, rid)

    return xn_final[inv]


# speedup vs baseline: 1.1791x; 1.1151x over previous
import jax
import jax.numpy as jnp
from jax.experimental import pallas as pl
from jax.experimental.pallas import tpu as pltpu

STEP_SIZE = 0.1
MAX_RESAMPLINGS = 50
N_ROWS = 16384
N_COLS = 128
BLOCK_ROWS = 1024
T1 = 12
B_BLOCK_ROWS = 1024


def _threefry2x32(k0, k1, x0, x1):
    def rotl(v, r):
        return (v << jnp.uint32(r)) | (v >> jnp.uint32(32 - r))

    ks2 = k0 ^ k1 ^ jnp.uint32(0x1BD11BDA)
    rots = ((13, 15, 26, 6), (17, 29, 16, 24))
    inject = ((k1, ks2, 1), (ks2, k0, 2), (k0, k1, 3), (k1, ks2, 4), (ks2, k0, 5))
    x0 = x0 + k0
    x1 = x1 + k1
    for g in range(5):
        for r in rots[g % 2]:
            x0 = x0 + x1
            x1 = rotl(x1, r)
            x1 = x0 ^ x1
        a, b, c = inject[g]
        x0 = x0 + a
        x1 = x1 + b + jnp.uint32(c)
    return x0, x1


def _uniform(k0, k1, idx):
    o0, o1 = _threefry2x32(k0, k1, jnp.zeros_like(idx), idx)
    bits = o0 ^ o1
    f = jax.lax.bitcast_convert_type(
        (bits >> jnp.uint32(9)) | jnp.uint32(0x3F800000), jnp.float32
    )
    return f - 1.0


def _one_iter(kd_ref, i, x, y, flat_ids, a, b, x_new, rej_f):
    reject = rej_f > 0.0
    smaller = x_new < x
    a = jnp.where(reject & smaller, x_new, a)
    b = jnp.where(reject & (~smaller), x_new, b)
    u = _uniform(kd_ref[4 + i, 0], kd_ref[4 + i, 1], flat_ids)
    x_new = jnp.where(reject, u * (b - a) + a, x_new)
    pot = -0.5 * jnp.sum(x_new * x_new, axis=-1, keepdims=True)
    rej_f = jnp.where(reject & (pot < y), 1.0, 0.0).astype(jnp.float32)
    return a, b, x_new, rej_f


def _phase_a(kd_ref, x_ref, xn_ref, a_ref, b_ref, y_ref, rej_ref):
    i_blk = pl.program_id(0)
    x = x_ref[...]
    rows, cols = x.shape

    row_in_blk = jax.lax.broadcasted_iota(jnp.uint32, (rows, 1), 0)
    row_ids = jnp.uint32(rows) * jnp.uint32(i_blk) + row_in_blk
    col_ids = jax.lax.broadcasted_iota(jnp.uint32, (rows, cols), 1)
    flat_ids = row_ids * jnp.uint32(cols) + col_ids

    pot_x = -0.5 * jnp.sum(x * x, axis=-1, keepdims=True)

    k0, k1 = kd_ref[0, 0], kd_ref[0, 1]
    y = jnp.log(1.0 - _uniform(k0, k1, row_ids)) + pot_x

    k0, k1 = kd_ref[2, 0], kd_ref[2, 1]
    sw = jnp.log(1.0 - _uniform(k0, k1, flat_ids)) / (-STEP_SIZE)
    a = x - sw / 2.0
    b = x + sw / 2.0

    k0, k1 = kd_ref[3, 0], kd_ref[3, 1]
    x_new = _uniform(k0, k1, flat_ids) * (b - a) + a
    pot = -0.5 * jnp.sum(x_new * x_new, axis=-1, keepdims=True)
    rej_f = jnp.where(pot < y, 1.0, 0.0).astype(jnp.float32)

    for i in range(T1):
        a, b, x_new, rej_f = _one_iter(kd_ref, i, x, y, flat_ids, a, b, x_new, rej_f)
    xn_ref[...] = x_new
    a_ref[...] = a
    b_ref[...] = b
    y_ref[...] = y
    rej_ref[...] = rej_f


def _phase_b(kd_ref, x_ref, xn_in_ref, a_ref, b_ref, y_ref, rej_ref, rid_ref,
             xn_out_ref):
    x = x_ref[...]
    rows, cols = x.shape
    row_ids = rid_ref[...].astype(jnp.uint32)
    col_ids = jax.lax.broadcasted_iota(jnp.uint32, (rows, cols), 1)
    flat_ids = row_ids * jnp.uint32(cols) + col_ids

    y = y_ref[...]
    a = a_ref[...]
    b = b_ref[...]
    x_new = xn_in_ref[...]
    rej_f = rej_ref[...]

    def cond(carry):
        i, _, _, _, rej_f = carry
        return jnp.logical_and(i < MAX_RESAMPLINGS, jnp.sum(rej_f) > 0.0)

    def step(carry):
        i, a, b, x_new, rej_f = carry
        a, b, x_new, rej_f = _one_iter(kd_ref, i, x, y, flat_ids, a, b, x_new, rej_f)
        a, b, x_new, rej_f = _one_iter(
            kd_ref, i + 1, x, y, flat_ids, a, b, x_new, rej_f
        )
        return i + 2, a, b, x_new, rej_f

    _, _, _, x_new, _ = jax.lax.while_loop(
        cond, step, (jnp.int32(T1), a, b, x_new, rej_f)
    )
    xn_out_ref[...] = x_new


def _row_spec(rows=BLOCK_ROWS):
    return pl.BlockSpec((rows, N_COLS), lambda i: (i, 0))


def _col_spec(rows=BLOCK_ROWS):
    return pl.BlockSpec((rows, 1), lambda i: (i, 0))


@jax.jit
def kernel(x):
    ks = jax.random.split(jax.random.key(1), 4 + MAX_RESAMPLINGS)
    kd = jax.random.key_data(ks).astype(jnp.uint32)

    mat = jax.ShapeDtypeStruct((N_ROWS, N_COLS), jnp.float32)
    col = jax.ShapeDtypeStruct((N_ROWS, 1), jnp.float32)

    x_new, a, b, y, rej = pl.pallas_call(
        _phase_a,
        grid=(N_ROWS // BLOCK_ROWS,),
        in_specs=[pl.BlockSpec(memory_space=pltpu.SMEM), _row_spec()],
        out_specs=[_row_spec(), _row_spec(), _row_spec(), _col_spec(), _col_spec()],
        out_shape=[mat, mat, mat, col, col],
        compiler_params=pltpu.CompilerParams(
            dimension_semantics=("arbitrary",),
        ),
    )(kd, x)

    order = jnp.argsort(-rej[:, 0]).astype(jnp.int32)
    inv = jnp.argsort(order).astype(jnp.int32)

    xs = x[order]
    xns = x_new[order]
    as_ = a[order]
    bs = b[order]
    ys = y[order]
    rejs = rej[order]
    rid = order[:, None]

    xn_final = pl.pallas_call(
        _phase_b,
        grid=(N_ROWS // B_BLOCK_ROWS,),
        in_specs=[
            pl.BlockSpec(memory_space=pltpu.SMEM),
            _row_spec(B_BLOCK_ROWS), _row_spec(B_BLOCK_ROWS),
            _row_spec(B_BLOCK_ROWS), _row_spec(B_BLOCK_ROWS),
            _col_spec(B_BLOCK_ROWS), _col_spec(B_BLOCK_ROWS),
            pl.BlockSpec((B_BLOCK_ROWS, 1), lambda i: (i, 0)),
        ],
        out_specs=_row_spec(B_BLOCK_ROWS),
        out_shape=mat,
        compiler_params=pltpu.CompilerParams(
            dimension_semantics=("arbitrary",),
        ),
    )(kd, xs, xns, as_, bs, ys, rejs, rid)

    return xn_final[inv]


# speedup vs baseline: 1.1875x; 1.0071x over previous
import jax
import jax.numpy as jnp
from jax.experimental import pallas as pl
from jax.experimental.pallas import tpu as pltpu

STEP_SIZE = 0.1
MAX_RESAMPLINGS = 50
N_ROWS = 16384
N_COLS = 128
BLOCK_ROWS = 1024
T1 = 12
B_BLOCK_ROWS = 1024


def _threefry2x32(k0, k1, x0, x1):
    def rotl(v, r):
        return (v << jnp.uint32(r)) | (v >> jnp.uint32(32 - r))

    ks2 = k0 ^ k1 ^ jnp.uint32(0x1BD11BDA)
    rots = ((13, 15, 26, 6), (17, 29, 16, 24))
    inject = ((k1, ks2, 1), (ks2, k0, 2), (k0, k1, 3), (k1, ks2, 4), (ks2, k0, 5))
    x0 = x0 + k0
    x1 = x1 + k1
    for g in range(5):
        for r in rots[g % 2]:
            x0 = x0 + x1
            x1 = rotl(x1, r)
            x1 = x0 ^ x1
        a, b, c = inject[g]
        x0 = x0 + a
        x1 = x1 + b + jnp.uint32(c)
    return x0, x1


def _uniform(k0, k1, idx):
    o0, o1 = _threefry2x32(k0, k1, jnp.zeros_like(idx), idx)
    bits = o0 ^ o1
    f = jax.lax.bitcast_convert_type(
        (bits >> jnp.uint32(9)) | jnp.uint32(0x3F800000), jnp.float32
    )
    return f - 1.0


def _one_iter(kd_ref, i, x, y, flat_ids, a, b, x_new, rej_f):
    reject = rej_f > 0.0
    smaller = x_new < x
    a = jnp.where(reject & smaller, x_new, a)
    b = jnp.where(reject & (~smaller), x_new, b)
    u = _uniform(kd_ref[4 + i, 0], kd_ref[4 + i, 1], flat_ids)
    x_new = jnp.where(reject, u * (b - a) + a, x_new)
    pot = -0.5 * jnp.sum(x_new * x_new, axis=-1, keepdims=True)
    rej_f = jnp.where(reject & (pot < y), 1.0, 0.0).astype(jnp.float32)
    return a, b, x_new, rej_f


def _phase_a(kd_ref, x_ref, st_ref, rej_ref):
    i_blk = pl.program_id(0)
    x = x_ref[...]
    rows, cols = x.shape

    row_in_blk = jax.lax.broadcasted_iota(jnp.uint32, (rows, 1), 0)
    row_ids = jnp.uint32(rows) * jnp.uint32(i_blk) + row_in_blk
    col_ids = jax.lax.broadcasted_iota(jnp.uint32, (rows, cols), 1)
    flat_ids = row_ids * jnp.uint32(cols) + col_ids

    pot_x = -0.5 * jnp.sum(x * x, axis=-1, keepdims=True)

    k0, k1 = kd_ref[0, 0], kd_ref[0, 1]
    y = jnp.log(1.0 - _uniform(k0, k1, row_ids)) + pot_x

    k0, k1 = kd_ref[2, 0], kd_ref[2, 1]
    sw = jnp.log(1.0 - _uniform(k0, k1, flat_ids)) / (-STEP_SIZE)
    a = x - sw / 2.0
    b = x + sw / 2.0

    k0, k1 = kd_ref[3, 0], kd_ref[3, 1]
    x_new = _uniform(k0, k1, flat_ids) * (b - a) + a
    pot = -0.5 * jnp.sum(x_new * x_new, axis=-1, keepdims=True)
    rej_f = jnp.where(pot < y, 1.0, 0.0).astype(jnp.float32)

    for i in range(T1):
        a, b, x_new, rej_f = _one_iter(kd_ref, i, x, y, flat_ids, a, b, x_new, rej_f)
    st_ref[:, 0:N_COLS] = x_new
    st_ref[:, N_COLS : 2 * N_COLS] = a
    st_ref[:, 2 * N_COLS : 3 * N_COLS] = b
    st_ref[:, 3 * N_COLS : 4 * N_COLS] = x
    rej_ref[...] = rej_f


def _phase_b(kd_ref, st_ref, rej_ref, rid_ref, xn_out_ref):
    x_new = st_ref[:, 0:N_COLS]
    a = st_ref[:, N_COLS : 2 * N_COLS]
    b = st_ref[:, 2 * N_COLS : 3 * N_COLS]
    x = st_ref[:, 3 * N_COLS : 4 * N_COLS]
    rows, cols = x.shape
    row_ids = rid_ref[...].astype(jnp.uint32)
    col_ids = jax.lax.broadcasted_iota(jnp.uint32, (rows, cols), 1)
    flat_ids = row_ids * jnp.uint32(cols) + col_ids

    pot_x = -0.5 * jnp.sum(x * x, axis=-1, keepdims=True)
    k0, k1 = kd_ref[0, 0], kd_ref[0, 1]
    y = jnp.log(1.0 - _uniform(k0, k1, row_ids)) + pot_x
    rej_f = rej_ref[...]

    def cond(carry):
        i, _, _, _, rej_f = carry
        return jnp.logical_and(i < MAX_RESAMPLINGS, jnp.sum(rej_f) > 0.0)

    def step(carry):
        i, a, b, x_new, rej_f = carry
        a, b, x_new, rej_f = _one_iter(kd_ref, i, x, y, flat_ids, a, b, x_new, rej_f)
        a, b, x_new, rej_f = _one_iter(
            kd_ref, i + 1, x, y, flat_ids, a, b, x_new, rej_f
        )
        return i + 2, a, b, x_new, rej_f

    _, _, _, x_new, _ = jax.lax.while_loop(
        cond, step, (jnp.int32(T1), a, b, x_new, rej_f)
    )
    xn_out_ref[...] = x_new


def _row_spec(rows=BLOCK_ROWS):
    return pl.BlockSpec((rows, N_COLS), lambda i: (i, 0))


def _col_spec(rows=BLOCK_ROWS):
    return pl.BlockSpec((rows, 1), lambda i: (i, 0))


@jax.jit
def kernel(x):
    ks = jax.random.split(jax.random.key(1), 4 + MAX_RESAMPLINGS)
    kd = jax.random.key_data(ks).astype(jnp.uint32)

    mat = jax.ShapeDtypeStruct((N_ROWS, N_COLS), jnp.float32)
    wide = jax.ShapeDtypeStruct((N_ROWS, 4 * N_COLS), jnp.float32)
    col = jax.ShapeDtypeStruct((N_ROWS, 1), jnp.float32)

    st, rej = pl.pallas_call(
        _phase_a,
        grid=(N_ROWS // BLOCK_ROWS,),
        in_specs=[pl.BlockSpec(memory_space=pltpu.SMEM), _row_spec()],
        out_specs=[
            pl.BlockSpec((BLOCK_ROWS, 4 * N_COLS), lambda i: (i, 0)),
            _col_spec(),
        ],
        out_shape=[wide, col],
        compiler_params=pltpu.CompilerParams(
            dimension_semantics=("arbitrary",),
        ),
    )(kd, x)

    order = jnp.argsort(-rej[:, 0]).astype(jnp.int32)
    inv = jnp.argsort(order).astype(jnp.int32)

    sts = st[order]
    rejs = rej[order]
    rid = order[:, None]

    xn_final = pl.pallas_call(
        _phase_b,
        grid=(N_ROWS // B_BLOCK_ROWS,),
        in_specs=[
            pl.BlockSpec(memory_space=pltpu.SMEM),
            pl.BlockSpec((B_BLOCK_ROWS, 4 * N_COLS), lambda i: (i, 0)),
            _col_spec(B_BLOCK_ROWS),
            pl.BlockSpec((B_BLOCK_ROWS, 1), lambda i: (i, 0)),
        ],
        out_specs=_row_spec(B_BLOCK_ROWS),
        out_shape=mat,
        compiler_params=pltpu.CompilerParams(
            dimension_semantics=("arbitrary",),
        ),
    )(kd, sts, rejs, rid)

    return xn_final[inv]
